# baseline (speedup 1.0000x reference)
import functools

import jax
import jax.numpy as jnp
from jax import lax
from jax.experimental import pallas as pl
from jax.experimental.pallas import tpu as pltpu
from jax.experimental.pallas import tpu_sc as plsc

N = 10000
D = 75
DP = 80
OUT = 128
E = 640000
EPS = 1e-5

NC = 2
NS = 16
NW = NC * NS
C = 128
CH = 159
NB = 3
K = 1
EP = NW * CH * C
OUTROWS = 10240
ZROWS = 640
PROWS = N // NS


def _spmm_body(s_hbm, src_hbm, dst_hbm, z_hbm, part_hbm,
               acc_sh, sidx, didx, rows, gsem, ssem):
  cid = lax.axis_index("c")
  sid = lax.axis_index("s")
  gw = cid * NS + sid

  pltpu.async_copy(src_hbm.at[gw], sidx, gsem.at[0])
  pltpu.async_copy(dst_hbm.at[gw], didx, gsem.at[1])
  for t in range(ZROWS // C):
    pltpu.async_copy(z_hbm, acc_sh.at[pl.ds(sid * ZROWS + t * C, C)],
                     ssem.at[t % NB])
  pltpu.make_async_copy(src_hbm.at[gw], sidx, gsem.at[0]).wait()
  pltpu.make_async_copy(dst_hbm.at[gw], didx, gsem.at[1]).wait()
  for t in range(ZROWS // C):
    pltpu.make_async_copy(z_hbm, acc_sh.at[pl.ds(0, C)], ssem.at[t % NB]).wait()
  plsc.subcore_barrier()

  def body(j, carry):
    for c in range(NB):
      ch = j * NB + c

      @pl.when(j >= 1)
      def _():
        pltpu.make_async_copy(rows.at[c], acc_sh.at[didx.at[0]],
                              ssem.at[c]).wait()

      pltpu.async_copy(s_hbm.at[sidx.at[ch]], rows.at[c], gsem.at[c])

      bb = (c + NB - K) % NB
      ch2 = ch - K

      @pl.when(ch2 >= 0)
      def _():
        pltpu.make_async_copy(s_hbm.at[sidx.at[0]], rows.at[bb],
                              gsem.at[bb]).wait()
        pltpu.async_copy(rows.at[bb], acc_sh.at[didx.at[ch2]], ssem.at[bb],
                         add=True)

    return carry

  lax.fori_loop(0, CH // NB, body, 0)

  for c in range(K):
    ch2 = CH - K + c
    bb = ch2 % NB
    pltpu.make_async_copy(s_hbm.at[sidx.at[0]], rows.at[bb],
                          gsem.at[bb]).wait()
    pltpu.async_copy(rows.at[bb], acc_sh.at[didx.at[ch2]], ssem.at[bb],
                     add=True)
  for b in range(NB):
    pltpu.make_async_copy(rows.at[b], acc_sh.at[didx.at[0]], ssem.at[b]).wait()
  plsc.subcore_barrier()

  pltpu.sync_copy(acc_sh.at[pl.ds(sid * ZROWS, ZROWS)],
                  part_hbm.at[cid, pl.ds(sid * ZROWS, ZROWS), pl.ds(0, DP)])


@functools.cache
def _mesh():
  return plsc.VectorSubcoreMesh(core_axis_name="c", subcore_axis_name="s",
                                num_cores=NC, num_subcores=NS)


@functools.cache
def _get_spmm():
  return pl.kernel(
      _spmm_body,
      out_type=jax.ShapeDtypeStruct((NC, OUTROWS, 128), jnp.float32),
      mesh=_mesh(),
      scratch_types=[
          pltpu.VMEM_SHARED((OUTROWS, DP), jnp.float32),
          pltpu.VMEM((CH, C), jnp.int32),
          pltpu.VMEM((CH, C), jnp.int32),
          pltpu.VMEM((NB, C, DP), jnp.float32),
          pltpu.SemaphoreType.DMA((NB,)),
          pltpu.SemaphoreType.DMA((NB,)),
      ],
      compiler_params=pltpu.CompilerParams(use_tc_tiling_on_sc=False),
  )


def _deg_body(dst_hbm, ones_hbm, z_hbm, degp_hbm,
              acc_sh, didx, ones_v, zbuf, sem):
  cid = lax.axis_index("c")
  sid = lax.axis_index("s")
  gw = cid * NS + sid

  pltpu.sync_copy(z_hbm, zbuf)
  pltpu.sync_copy(zbuf, acc_sh.at[pl.ds(sid * ZROWS, ZROWS)])
  pltpu.sync_copy(ones_hbm, ones_v)
  plsc.subcore_barrier()

  pltpu.sync_copy(dst_hbm.at[gw], didx)

  for c in range(NB):
    pltpu.async_copy(ones_v, acc_sh.at[didx.at[c]], sem, add=True)

  def body(k, carry):
    pltpu.make_async_copy(ones_v, acc_sh.at[didx.at[0]], sem).wait()
    pltpu.async_copy(ones_v, acc_sh.at[didx.at[k]], sem, add=True)
    return carry

  lax.fori_loop(NB, CH, body, 0)
  for c in range(NB):
    pltpu.make_async_copy(ones_v, acc_sh.at[didx.at[0]], sem).wait()
  plsc.subcore_barrier()

  pltpu.sync_copy(acc_sh.at[pl.ds(sid * ZROWS, ZROWS)],
                  degp_hbm.at[cid, pl.ds(sid * ZROWS, ZROWS)])


@functools.cache
def _get_deg():
  return pl.kernel(
      _deg_body,
      out_type=jax.ShapeDtypeStruct((NC, OUTROWS, 16), jnp.float32),
      mesh=_mesh(),
      scratch_types=[
          pltpu.VMEM_SHARED((OUTROWS, 16), jnp.float32),
          pltpu.VMEM((CH, C), jnp.int32),
          pltpu.VMEM((C, 16), jnp.float32),
          pltpu.VMEM((ZROWS, 16), jnp.float32),
          pltpu.SemaphoreType.DMA,
      ],
      compiler_params=pltpu.CompilerParams(use_tc_tiling_on_sc=False),
  )


def _pad128(x):
  return jnp.pad(x, ((0, 0), (0, 128 - x.shape[1])))


def _prep_body(degp_ref, h0_ref, dinv_ref, s0_ref):
  deg = degp_ref[0, :N, 0:1] + degp_ref[1, :N, 0:1] + 1.0
  dinv = lax.rsqrt(deg)
  dinv_ref[...] = dinv
  s0_ref[...] = h0_ref[...] * dinv


def _layer_body(p_ref, s_ref, dinv_ref, w_ref, b_ref, g_ref, bt_ref, o_ref):
  dinv = dinv_ref[...]
  z = (p_ref[0, :N, :DP] + p_ref[1, :N, :DP] + s_ref[...]) * dinv
  h = jnp.dot(z, w_ref[...], preferred_element_type=jnp.float32) + b_ref[...]
  mu = jnp.mean(h, axis=0, keepdims=True)
  var = jnp.mean((h - mu) * (h - mu), axis=0, keepdims=True)
  hn = (h - mu) * lax.rsqrt(var + EPS) * g_ref[...] + bt_ref[...]
  o_ref[...] = jnp.maximum(hn, 0.0) * dinv


def _final_body(p_ref, s_ref, dinv_ref, wmu_ref, bmu_ref, wls_ref, bls_ref,
                mu_ref, ls_ref):
  z = (p_ref[0, :N, :DP] + p_ref[1, :N, :DP] + s_ref[...]) * dinv_ref[...]
  mu_ref[...] = (
      jnp.dot(z, wmu_ref[...], preferred_element_type=jnp.float32)
      + bmu_ref[...])
  ls_ref[...] = (
      jnp.dot(z, wls_ref[...], preferred_element_type=jnp.float32)
      + bls_ref[...])


_f32 = jnp.float32
_prep = pl.pallas_call(
    _prep_body,
    out_shape=(jax.ShapeDtypeStruct((N, 1), _f32),
               jax.ShapeDtypeStruct((N, DP), _f32)),
)
_layer = pl.pallas_call(
    _layer_body,
    out_shape=jax.ShapeDtypeStruct((N, DP), _f32),
)
_final = pl.pallas_call(
    _final_body,
    out_shape=(jax.ShapeDtypeStruct((N, OUT), _f32),
               jax.ShapeDtypeStruct((N, OUT), _f32)),
)


def _pad_w(w):
  return jnp.pad(w, ((0, DP - D), (0, max(0, DP - w.shape[1]))))


def _pad_row(v, width):
  return jnp.pad(v, (0, width - v.shape[0]))[None, :]


@jax.jit
def kernel(x, edge_index, emb, W0, b0, g0, bt0, W1, b1, g1, bt1,
           W2, b2, g2, bt2, W3, b3, g3, bt3, Wmu, bmu, Wls, bls):
  h0 = jnp.pad(jnp.take(emb, x, axis=0), ((0, 0), (0, DP - D)))

  pad = EP - E
  pad_idx = jnp.arange(pad, dtype=jnp.int32)
  srcp = jnp.concatenate([edge_index[0], pad_idx % N]).reshape(NW, CH, C)
  dstp = jnp.concatenate(
      [edge_index[1], N + pad_idx % (OUTROWS - N)]).reshape(NW, CH, C)

  zrows = jnp.zeros((C, DP), _f32)
  ones16 = jnp.ones((C, 16), _f32)
  z16 = jnp.zeros((ZROWS, 16), _f32)

  degp = _get_deg()(dstp, ones16, z16)
  dinv, s = _prep(degp, h0)

  ws = [(_pad_w(W0), _pad_row(b0, DP), _pad_row(g0, DP), _pad_row(bt0, DP)),
        (_pad_w(W1), _pad_row(b1, DP), _pad_row(g1, DP), _pad_row(bt1, DP)),
        (_pad_w(W2), _pad_row(b2, DP), _pad_row(g2, DP), _pad_row(bt2, DP)),
        (_pad_w(W3), _pad_row(b3, DP), _pad_row(g3, DP), _pad_row(bt3, DP))]
  spmm = _get_spmm()
  for w, b, g, bt in ws:
    part = spmm(s, srcp, dstp, zrows)
    s = _layer(part, s, dinv, w, b, g, bt)

  part = spmm(s, srcp, dstp, zrows)
  mu, ls = _final(part, s, dinv, _pad_w(Wmu), _pad_row(bmu, OUT),
                  _pad_w(Wls), _pad_row(bls, OUT))
  return mu, ls


# speedup vs baseline: 1.0471x; 1.0471x over previous
import functools

import jax
import jax.numpy as jnp
from jax import lax
from jax.experimental import pallas as pl
from jax.experimental.pallas import tpu as pltpu
from jax.experimental.pallas import tpu_sc as plsc

N = 10000
D = 75
DP = 80
OUT = 128
E = 640000
EPS = 1e-5

NC = 2
NS = 16
NW = NC * NS
C = 128
CH = 159
NB = 3
K = 1
EP = NW * CH * C
OUTROWS = 10240
ZROWS = 640
PROWS = N // NS


def _spmm_body(s_hbm, src_hbm, dst_hbm, z_hbm, part_hbm,
               acc_sh, sidx, didx, rows, gsem, ssem):
  cid = lax.axis_index("c")
  sid = lax.axis_index("s")
  gw = cid * NS + sid

  pltpu.async_copy(src_hbm.at[gw], sidx, gsem.at[0])
  pltpu.async_copy(dst_hbm.at[gw], didx, gsem.at[1])
  pltpu.sync_copy(z_hbm, rows.at[0])
  for t in range(ZROWS // C):
    pltpu.async_copy(rows.at[0], acc_sh.at[pl.ds(sid * ZROWS + t * C, C)],
                     ssem.at[t % NB])
  pltpu.make_async_copy(src_hbm.at[gw], sidx, gsem.at[0]).wait()
  pltpu.make_async_copy(dst_hbm.at[gw], didx, gsem.at[1]).wait()
  for t in range(ZROWS // C):
    pltpu.make_async_copy(rows.at[0], acc_sh.at[pl.ds(0, C)],
                          ssem.at[t % NB]).wait()
  plsc.subcore_barrier()

  def body(j, carry):
    for c in range(NB):
      ch = j * NB + c

      @pl.when(j >= 1)
      def _():
        pltpu.make_async_copy(rows.at[c], acc_sh.at[didx.at[0]],
                              ssem.at[c]).wait()

      pltpu.async_copy(s_hbm.at[sidx.at[ch]], rows.at[c], gsem.at[c])

      bb = (c + NB - K) % NB
      ch2 = ch - K

      @pl.when(ch2 >= 0)
      def _():
        pltpu.make_async_copy(s_hbm.at[sidx.at[0]], rows.at[bb],
                              gsem.at[bb]).wait()
        pltpu.async_copy(rows.at[bb], acc_sh.at[didx.at[ch2]], ssem.at[bb],
                         add=True)

    return carry

  lax.fori_loop(0, CH // NB, body, 0)

  for c in range(K):
    ch2 = CH - K + c
    bb = ch2 % NB
    pltpu.make_async_copy(s_hbm.at[sidx.at[0]], rows.at[bb],
                          gsem.at[bb]).wait()
    pltpu.async_copy(rows.at[bb], acc_sh.at[didx.at[ch2]], ssem.at[bb],
                     add=True)
  for b in range(NB):
    pltpu.make_async_copy(rows.at[b], acc_sh.at[didx.at[0]], ssem.at[b]).wait()
  plsc.subcore_barrier()

  pltpu.sync_copy(acc_sh.at[pl.ds(sid * ZROWS, ZROWS)],
                  part_hbm.at[cid, pl.ds(sid * ZROWS, ZROWS), pl.ds(0, DP)])


@functools.cache
def _mesh():
  return plsc.VectorSubcoreMesh(core_axis_name="c", subcore_axis_name="s",
                                num_cores=NC, num_subcores=NS)


@functools.cache
def _get_spmm():
  return pl.kernel(
      _spmm_body,
      out_type=jax.ShapeDtypeStruct((NC, OUTROWS, 128), jnp.float32),
      mesh=_mesh(),
      scratch_types=[
          pltpu.VMEM_SHARED((OUTROWS, DP), jnp.float32),
          pltpu.VMEM((CH, C), jnp.int32),
          pltpu.VMEM((CH, C), jnp.int32),
          pltpu.VMEM((NB, C, DP), jnp.float32),
          pltpu.SemaphoreType.DMA((NB,)),
          pltpu.SemaphoreType.DMA((NB,)),
      ],
      compiler_params=pltpu.CompilerParams(use_tc_tiling_on_sc=False),
  )


def _deg_body(dst_hbm, ones_hbm, z_hbm, degp_hbm,
              acc_sh, didx, ones_v, zbuf, sem):
  cid = lax.axis_index("c")
  sid = lax.axis_index("s")
  gw = cid * NS + sid

  pltpu.sync_copy(z_hbm, zbuf)
  pltpu.sync_copy(zbuf, acc_sh.at[pl.ds(sid * ZROWS, ZROWS)])
  pltpu.sync_copy(ones_hbm, ones_v)
  plsc.subcore_barrier()

  pltpu.sync_copy(dst_hbm.at[gw], didx)

  for c in range(NB):
    pltpu.async_copy(ones_v, acc_sh.at[didx.at[c]], sem, add=True)

  def body(k, carry):
    pltpu.make_async_copy(ones_v, acc_sh.at[didx.at[0]], sem).wait()
    pltpu.async_copy(ones_v, acc_sh.at[didx.at[k]], sem, add=True)
    return carry

  lax.fori_loop(NB, CH, body, 0)
  for c in range(NB):
    pltpu.make_async_copy(ones_v, acc_sh.at[didx.at[0]], sem).wait()
  plsc.subcore_barrier()

  pltpu.sync_copy(acc_sh.at[pl.ds(sid * ZROWS, ZROWS)],
                  degp_hbm.at[cid, pl.ds(sid * ZROWS, ZROWS)])


@functools.cache
def _get_deg():
  return pl.kernel(
      _deg_body,
      out_type=jax.ShapeDtypeStruct((NC, OUTROWS, 16), jnp.float32),
      mesh=_mesh(),
      scratch_types=[
          pltpu.VMEM_SHARED((OUTROWS, 16), jnp.float32),
          pltpu.VMEM((CH, C), jnp.int32),
          pltpu.VMEM((C, 16), jnp.float32),
          pltpu.VMEM((ZROWS, 16), jnp.float32),
          pltpu.SemaphoreType.DMA,
      ],
      compiler_params=pltpu.CompilerParams(use_tc_tiling_on_sc=False),
  )


def _pad128(x):
  return jnp.pad(x, ((0, 0), (0, 128 - x.shape[1])))


def _prep_body(degp_ref, h0_ref, dinv_ref, s0_ref):
  deg = degp_ref[0, :N, 0:1] + degp_ref[1, :N, 0:1] + 1.0
  dinv = lax.rsqrt(deg)
  dinv_ref[...] = dinv
  s0_ref[...] = h0_ref[...] * dinv


def _layer_body(p_ref, s_ref, dinv_ref, w_ref, b_ref, g_ref, bt_ref, o_ref):
  dinv = dinv_ref[...]
  z = (p_ref[0, :N, :DP] + p_ref[1, :N, :DP] + s_ref[...]) * dinv
  h = jnp.dot(z, w_ref[...], preferred_element_type=jnp.float32) + b_ref[...]
  mu = jnp.mean(h, axis=0, keepdims=True)
  var = jnp.mean((h - mu) * (h - mu), axis=0, keepdims=True)
  hn = (h - mu) * lax.rsqrt(var + EPS) * g_ref[...] + bt_ref[...]
  o_ref[...] = jnp.maximum(hn, 0.0) * dinv


def _final_body(p_ref, s_ref, dinv_ref, wmu_ref, bmu_ref, wls_ref, bls_ref,
                mu_ref, ls_ref):
  z = (p_ref[0, :N, :DP] + p_ref[1, :N, :DP] + s_ref[...]) * dinv_ref[...]
  mu_ref[...] = (
      jnp.dot(z, wmu_ref[...], preferred_element_type=jnp.float32)
      + bmu_ref[...])
  ls_ref[...] = (
      jnp.dot(z, wls_ref[...], preferred_element_type=jnp.float32)
      + bls_ref[...])


_f32 = jnp.float32
_prep = pl.pallas_call(
    _prep_body,
    out_shape=(jax.ShapeDtypeStruct((N, 1), _f32),
               jax.ShapeDtypeStruct((N, DP), _f32)),
)
_layer = pl.pallas_call(
    _layer_body,
    out_shape=jax.ShapeDtypeStruct((N, DP), _f32),
)
_final = pl.pallas_call(
    _final_body,
    out_shape=(jax.ShapeDtypeStruct((N, OUT), _f32),
               jax.ShapeDtypeStruct((N, OUT), _f32)),
)


def _pad_w(w):
  return jnp.pad(w, ((0, DP - D), (0, max(0, DP - w.shape[1]))))


def _pad_row(v, width):
  return jnp.pad(v, (0, width - v.shape[0]))[None, :]


@jax.jit
def kernel(x, edge_index, emb, W0, b0, g0, bt0, W1, b1, g1, bt1,
           W2, b2, g2, bt2, W3, b3, g3, bt3, Wmu, bmu, Wls, bls):
  h0 = jnp.pad(jnp.take(emb, x, axis=0), ((0, 0), (0, DP - D)))

  pad = EP - E
  pad_idx = jnp.arange(pad, dtype=jnp.int32)
  srcp = jnp.concatenate([edge_index[0], pad_idx % N]).reshape(NW, CH, C)
  dstp = jnp.concatenate(
      [edge_index[1], N + pad_idx % (OUTROWS - N)]).reshape(NW, CH, C)

  zrows = jnp.zeros((C, DP), _f32)
  ones16 = jnp.ones((C, 16), _f32)
  z16 = jnp.zeros((ZROWS, 16), _f32)

  degp = _get_deg()(dstp, ones16, z16)
  dinv, s = _prep(degp, h0)

  ws = [(_pad_w(W0), _pad_row(b0, DP), _pad_row(g0, DP), _pad_row(bt0, DP)),
        (_pad_w(W1), _pad_row(b1, DP), _pad_row(g1, DP), _pad_row(bt1, DP)),
        (_pad_w(W2), _pad_row(b2, DP), _pad_row(g2, DP), _pad_row(bt2, DP)),
        (_pad_w(W3), _pad_row(b3, DP), _pad_row(g3, DP), _pad_row(bt3, DP))]
  spmm = _get_spmm()
  for w, b, g, bt in ws:
    part = spmm(s, srcp, dstp, zrows)
    s = _layer(part, s, dinv, w, b, g, bt)

  part = spmm(s, srcp, dstp, zrows)
  mu, ls = _final(part, s, dinv, _pad_w(Wmu), _pad_row(bmu, OUT),
                  _pad_w(Wls), _pad_row(bls, OUT))
  return mu, ls


# speedup vs baseline: 1.0688x; 1.0208x over previous
import functools

import jax
import jax.numpy as jnp
from jax import lax
from jax.experimental import pallas as pl
from jax.experimental.pallas import tpu as pltpu
from jax.experimental.pallas import tpu_sc as plsc

N = 10000
D = 75
DP = 80
OUT = 128
E = 640000
EPS = 1e-5

NC = 2
NS = 16
NW = NC * NS
C = 128
NCH = E // C
CHW = NCH // NW
REM = NCH % NW
CH = CHW + 1
NB = 3
K = 1
OUTROWS = 10240
ZROWS = 640
PROWS = N // NS


def _spmm_body(s_hbm, src_hbm, dst_hbm, z_hbm, part_hbm,
               acc_sh, sidx, didx, rows, gsem, ssem):
  cid = lax.axis_index("c")
  sid = lax.axis_index("s")
  gw = cid * NS + sid
  nch = CHW + jnp.where(gw < REM, 1, 0)
  start = CHW * gw + jnp.minimum(gw, REM)

  @pl.when(gw < REM)
  def _():
    pltpu.async_copy(src_hbm.at[pl.ds(start, CH)], sidx, gsem.at[0])
    pltpu.async_copy(dst_hbm.at[pl.ds(start, CH)], didx, gsem.at[1])

  @pl.when(gw >= REM)
  def _():
    pltpu.async_copy(src_hbm.at[pl.ds(start, CHW)], sidx.at[pl.ds(0, CHW)],
                     gsem.at[0])
    pltpu.async_copy(dst_hbm.at[pl.ds(start, CHW)], didx.at[pl.ds(0, CHW)],
                     gsem.at[1])

  pltpu.sync_copy(z_hbm, rows.at[0])
  for t in range(ZROWS // C):
    pltpu.async_copy(rows.at[0], acc_sh.at[pl.ds(sid * ZROWS + t * C, C)],
                     ssem.at[t % NB])

  @pl.when(gw < REM)
  def _():
    pltpu.make_async_copy(src_hbm.at[pl.ds(0, CH)], sidx, gsem.at[0]).wait()
    pltpu.make_async_copy(dst_hbm.at[pl.ds(0, CH)], didx, gsem.at[1]).wait()

  @pl.when(gw >= REM)
  def _():
    pltpu.make_async_copy(src_hbm.at[pl.ds(0, CHW)], sidx.at[pl.ds(0, CHW)],
                          gsem.at[0]).wait()
    pltpu.make_async_copy(dst_hbm.at[pl.ds(0, CHW)], didx.at[pl.ds(0, CHW)],
                          gsem.at[1]).wait()

  for t in range(ZROWS // C):
    pltpu.make_async_copy(rows.at[0], acc_sh.at[pl.ds(0, C)],
                          ssem.at[t % NB]).wait()
  plsc.subcore_barrier()

  def body(j, carry):
    for c in range(NB):
      m = j * NB + c

      @pl.when((m >= NB) & (m < nch))
      def _():
        pltpu.make_async_copy(rows.at[c], acc_sh.at[didx.at[0]],
                              ssem.at[c]).wait()

      @pl.when(m < nch)
      def _():
        pltpu.async_copy(s_hbm.at[sidx.at[m]], rows.at[c], gsem.at[c])

      bb = (c + NB - K) % NB
      x = m - K

      @pl.when((x >= 0) & (x < nch))
      def _():
        pltpu.make_async_copy(s_hbm.at[sidx.at[0]], rows.at[bb],
                              gsem.at[bb]).wait()
        pltpu.async_copy(rows.at[bb], acc_sh.at[didx.at[x]], ssem.at[bb],
                         add=True)

    return carry

  lax.fori_loop(0, (CH + K + NB - 1) // NB, body, 0)

  for b in range(NB):
    pltpu.make_async_copy(rows.at[b], acc_sh.at[didx.at[0]], ssem.at[b]).wait()
  plsc.subcore_barrier()

  pltpu.sync_copy(acc_sh.at[pl.ds(sid * ZROWS, ZROWS)],
                  part_hbm.at[cid, pl.ds(sid * ZROWS, ZROWS), pl.ds(0, DP)])


@functools.cache
def _mesh():
  return plsc.VectorSubcoreMesh(core_axis_name="c", subcore_axis_name="s",
                                num_cores=NC, num_subcores=NS)


@functools.cache
def _get_spmm():
  return pl.kernel(
      _spmm_body,
      out_type=jax.ShapeDtypeStruct((NC, OUTROWS, 128), jnp.float32),
      mesh=_mesh(),
      scratch_types=[
          pltpu.VMEM_SHARED((OUTROWS, DP), jnp.float32),
          pltpu.VMEM((CH, C), jnp.int32),
          pltpu.VMEM((CH, C), jnp.int32),
          pltpu.VMEM((NB, C, DP), jnp.float32),
          pltpu.SemaphoreType.DMA((NB,)),
          pltpu.SemaphoreType.DMA((NB,)),
      ],
      compiler_params=pltpu.CompilerParams(use_tc_tiling_on_sc=False),
  )


def _deg_body(dst_hbm, ones_hbm, z_hbm, degp_hbm,
              acc_sh, didx, ones_v, zbuf, sem):
  cid = lax.axis_index("c")
  sid = lax.axis_index("s")
  gw = cid * NS + sid
  nch = CHW + jnp.where(gw < REM, 1, 0)
  start = CHW * gw + jnp.minimum(gw, REM)

  pltpu.sync_copy(z_hbm, zbuf)
  pltpu.sync_copy(zbuf, acc_sh.at[pl.ds(sid * ZROWS, ZROWS)])
  pltpu.sync_copy(ones_hbm, ones_v)
  plsc.subcore_barrier()

  @pl.when(gw < REM)
  def _():
    pltpu.sync_copy(dst_hbm.at[pl.ds(start, CH)], didx)

  @pl.when(gw >= REM)
  def _():
    pltpu.sync_copy(dst_hbm.at[pl.ds(start, CHW)], didx.at[pl.ds(0, CHW)])

  nf = 8

  def body(k, carry):
    @pl.when(k >= nf)
    def _():
      pltpu.make_async_copy(ones_v, acc_sh.at[didx.at[0]], sem).wait()

    pltpu.async_copy(ones_v, acc_sh.at[didx.at[k]], sem, add=True)
    return carry

  lax.fori_loop(0, nch, body, 0)
  for c in range(nf):
    pltpu.make_async_copy(ones_v, acc_sh.at[didx.at[0]], sem).wait()
  plsc.subcore_barrier()

  pltpu.sync_copy(acc_sh.at[pl.ds(sid * ZROWS, ZROWS)],
                  degp_hbm.at[cid, pl.ds(sid * ZROWS, ZROWS)])


@functools.cache
def _get_deg():
  return pl.kernel(
      _deg_body,
      out_type=jax.ShapeDtypeStruct((NC, OUTROWS, 16), jnp.float32),
      mesh=_mesh(),
      scratch_types=[
          pltpu.VMEM_SHARED((OUTROWS, 16), jnp.float32),
          pltpu.VMEM((CH, C), jnp.int32),
          pltpu.VMEM((C, 16), jnp.float32),
          pltpu.VMEM((ZROWS, 16), jnp.float32),
          pltpu.SemaphoreType.DMA,
      ],
      compiler_params=pltpu.CompilerParams(use_tc_tiling_on_sc=False),
  )


def _pad128(x):
  return jnp.pad(x, ((0, 0), (0, 128 - x.shape[1])))


def _prep_body(degp_ref, h0_ref, dinv_ref, s0_ref):
  deg = degp_ref[0, :N, 0:1] + degp_ref[1, :N, 0:1] + 1.0
  dinv = lax.rsqrt(deg)
  dinv_ref[...] = dinv
  s0_ref[...] = h0_ref[...] * dinv


def _layer_body(p_ref, s_ref, dinv_ref, w_ref, b_ref, g_ref, bt_ref, o_ref):
  dinv = dinv_ref[...]
  z = (p_ref[0, :N, :DP] + p_ref[1, :N, :DP] + s_ref[...]) * dinv
  h = jnp.dot(z, w_ref[...], preferred_element_type=jnp.float32) + b_ref[...]
  mu = jnp.mean(h, axis=0, keepdims=True)
  var = jnp.mean((h - mu) * (h - mu), axis=0, keepdims=True)
  hn = (h - mu) * lax.rsqrt(var + EPS) * g_ref[...] + bt_ref[...]
  o_ref[...] = jnp.maximum(hn, 0.0) * dinv


def _final_body(p_ref, s_ref, dinv_ref, wmu_ref, bmu_ref, wls_ref, bls_ref,
                mu_ref, ls_ref):
  z = (p_ref[0, :N, :DP] + p_ref[1, :N, :DP] + s_ref[...]) * dinv_ref[...]
  mu_ref[...] = (
      jnp.dot(z, wmu_ref[...], preferred_element_type=jnp.float32)
      + bmu_ref[...])
  ls_ref[...] = (
      jnp.dot(z, wls_ref[...], preferred_element_type=jnp.float32)
      + bls_ref[...])


_f32 = jnp.float32
_prep = pl.pallas_call(
    _prep_body,
    out_shape=(jax.ShapeDtypeStruct((N, 1), _f32),
               jax.ShapeDtypeStruct((N, DP), _f32)),
)
_layer = pl.pallas_call(
    _layer_body,
    out_shape=jax.ShapeDtypeStruct((N, DP), _f32),
)
_final = pl.pallas_call(
    _final_body,
    out_shape=(jax.ShapeDtypeStruct((N, OUT), _f32),
               jax.ShapeDtypeStruct((N, OUT), _f32)),
)


def _pad_w(w):
  return jnp.pad(w, ((0, DP - D), (0, max(0, DP - w.shape[1]))))


def _pad_row(v, width):
  return jnp.pad(v, (0, width - v.shape[0]))[None, :]


@jax.jit
def kernel(x, edge_index, emb, W0, b0, g0, bt0, W1, b1, g1, bt1,
           W2, b2, g2, bt2, W3, b3, g3, bt3, Wmu, bmu, Wls, bls):
  h0 = jnp.pad(jnp.take(emb, x, axis=0), ((0, 0), (0, DP - D)))

  srcp = edge_index[0].reshape(NCH, C)
  dstp = edge_index[1].reshape(NCH, C)

  zrows = jnp.zeros((C, DP), _f32)
  ones16 = jnp.ones((C, 16), _f32)
  z16 = jnp.zeros((ZROWS, 16), _f32)

  degp = _get_deg()(dstp, ones16, z16)
  dinv, s = _prep(degp, h0)

  ws = [(_pad_w(W0), _pad_row(b0, DP), _pad_row(g0, DP), _pad_row(bt0, DP)),
        (_pad_w(W1), _pad_row(b1, DP), _pad_row(g1, DP), _pad_row(bt1, DP)),
        (_pad_w(W2), _pad_row(b2, DP), _pad_row(g2, DP), _pad_row(bt2, DP)),
        (_pad_w(W3), _pad_row(b3, DP), _pad_row(g3, DP), _pad_row(bt3, DP))]
  spmm = _get_spmm()
  for w, b, g, bt in ws:
    part = spmm(s, srcp, dstp, zrows)
    s = _layer(part, s, dinv, w, b, g, bt)

  part = spmm(s, srcp, dstp, zrows)
  mu, ls = _final(part, s, dinv, _pad_w(Wmu), _pad_row(bmu, OUT),
                  _pad_w(Wls), _pad_row(bls, OUT))
  return mu, ls


# speedup vs baseline: 1.0693x; 1.0004x over previous
import functools

import jax
import jax.numpy as jnp
from jax import lax
from jax.experimental import pallas as pl
from jax.experimental.pallas import tpu as pltpu
from jax.experimental.pallas import tpu_sc as plsc

N = 10000
D = 75
DP = 80
OUT = 128
E = 640000
EPS = 1e-5

NC = 2
NS = 16
NW = NC * NS
C = 128
NCH = E // C
CHW = NCH // NW
REM = NCH % NW
CH = CHW + 1
NB = 3
K = 1
OUTROWS = 10240
ZROWS = 640
PROWS = N // NS


def _spmm_body(s_hbm, src_hbm, dst_hbm, z_hbm, part_hbm,
               acc_sh, sidx, didx, rows, gsem, ssem):
  cid = lax.axis_index("c")
  sid = lax.axis_index("s")
  gw = cid * NS + sid
  nch = CHW + jnp.where(gw < REM, 1, 0)
  start = CHW * gw + jnp.minimum(gw, REM)

  @pl.when(gw < REM)
  def _():
    pltpu.async_copy(src_hbm.at[pl.ds(start, CH)], sidx, gsem.at[0])
    pltpu.async_copy(dst_hbm.at[pl.ds(start, CH)], didx, gsem.at[1])

  @pl.when(gw >= REM)
  def _():
    pltpu.async_copy(src_hbm.at[pl.ds(start, CHW)], sidx.at[pl.ds(0, CHW)],
                     gsem.at[0])
    pltpu.async_copy(dst_hbm.at[pl.ds(start, CHW)], didx.at[pl.ds(0, CHW)],
                     gsem.at[1])

  pltpu.sync_copy(z_hbm, rows.at[0])
  for t in range(ZROWS // C):
    pltpu.async_copy(rows.at[0], acc_sh.at[pl.ds(sid * ZROWS + t * C, C)],
                     ssem.at[t % NB])

  @pl.when(gw < REM)
  def _():
    pltpu.make_async_copy(src_hbm.at[pl.ds(0, CH)], sidx, gsem.at[0]).wait()
    pltpu.make_async_copy(dst_hbm.at[pl.ds(0, CH)], didx, gsem.at[1]).wait()

  @pl.when(gw >= REM)
  def _():
    pltpu.make_async_copy(src_hbm.at[pl.ds(0, CHW)], sidx.at[pl.ds(0, CHW)],
                          gsem.at[0]).wait()
    pltpu.make_async_copy(dst_hbm.at[pl.ds(0, CHW)], didx.at[pl.ds(0, CHW)],
                          gsem.at[1]).wait()

  for t in range(ZROWS // C):
    pltpu.make_async_copy(rows.at[0], acc_sh.at[pl.ds(0, C)],
                          ssem.at[t % NB]).wait()
  plsc.subcore_barrier()

  def body(j, carry):
    for c in range(NB):
      m = j * NB + c

      @pl.when((m >= NB) & (m < nch))
      def _():
        pltpu.make_async_copy(rows.at[c], acc_sh.at[didx.at[0]],
                              ssem.at[c]).wait()

      @pl.when(m < nch)
      def _():
        pltpu.async_copy(s_hbm.at[sidx.at[m]], rows.at[c], gsem.at[c])

      bb = (c + NB - K) % NB
      x = m - K

      @pl.when((x >= 0) & (x < nch))
      def _():
        pltpu.make_async_copy(s_hbm.at[sidx.at[0]], rows.at[bb],
                              gsem.at[bb]).wait()
        pltpu.async_copy(rows.at[bb], acc_sh.at[didx.at[x]], ssem.at[bb],
                         add=True)

    return carry

  lax.fori_loop(0, (CH + K + NB - 1) // NB, body, 0)

  for b in range(NB):
    pltpu.make_async_copy(rows.at[b], acc_sh.at[didx.at[0]], ssem.at[b]).wait()
  plsc.subcore_barrier()

  pltpu.sync_copy(acc_sh.at[pl.ds(sid * ZROWS, ZROWS)],
                  part_hbm.at[cid, pl.ds(sid * ZROWS, ZROWS), pl.ds(0, DP)])


@functools.cache
def _mesh():
  return plsc.VectorSubcoreMesh(core_axis_name="c", subcore_axis_name="s",
                                num_cores=NC, num_subcores=NS)


@functools.cache
def _get_spmm():
  return pl.kernel(
      _spmm_body,
      out_type=jax.ShapeDtypeStruct((NC, OUTROWS, 128), jnp.float32),
      mesh=_mesh(),
      scratch_types=[
          pltpu.VMEM_SHARED((OUTROWS, DP), jnp.float32),
          pltpu.VMEM((CH, C), jnp.int32),
          pltpu.VMEM((CH, C), jnp.int32),
          pltpu.VMEM((NB, C, DP), jnp.float32),
          pltpu.SemaphoreType.DMA((NB,)),
          pltpu.SemaphoreType.DMA((NB,)),
      ],
      compiler_params=pltpu.CompilerParams(use_tc_tiling_on_sc=False),
  )


def _deg_body(dst_hbm, ones_hbm, z_hbm, degp_hbm,
              acc_sh, didx, ones_v, zbuf, sem):
  cid = lax.axis_index("c")
  sid = lax.axis_index("s")
  gw = cid * NS + sid
  nch = CHW + jnp.where(gw < REM, 1, 0)
  start = CHW * gw + jnp.minimum(gw, REM)

  pltpu.sync_copy(z_hbm, zbuf)
  pltpu.sync_copy(zbuf, acc_sh.at[pl.ds(sid * ZROWS, ZROWS)])
  pltpu.sync_copy(ones_hbm, ones_v)
  plsc.subcore_barrier()

  @pl.when(gw < REM)
  def _():
    pltpu.sync_copy(dst_hbm.at[pl.ds(start, CH)], didx)

  @pl.when(gw >= REM)
  def _():
    pltpu.sync_copy(dst_hbm.at[pl.ds(start, CHW)], didx.at[pl.ds(0, CHW)])

  nf = 8

  def body(k, carry):
    @pl.when(k >= nf)
    def _():
      pltpu.make_async_copy(ones_v, acc_sh.at[didx.at[0]], sem).wait()

    pltpu.async_copy(ones_v, acc_sh.at[didx.at[k]], sem, add=True)
    return carry

  lax.fori_loop(0, nch, body, 0)
  for c in range(nf):
    pltpu.make_async_copy(ones_v, acc_sh.at[didx.at[0]], sem).wait()
  plsc.subcore_barrier()

  pltpu.sync_copy(acc_sh.at[pl.ds(sid * ZROWS, ZROWS)],
                  degp_hbm.at[cid, pl.ds(sid * ZROWS, ZROWS)])


@functools.cache
def _get_deg():
  return pl.kernel(
      _deg_body,
      out_type=jax.ShapeDtypeStruct((NC, OUTROWS, 16), jnp.float32),
      mesh=_mesh(),
      scratch_types=[
          pltpu.VMEM_SHARED((OUTROWS, 16), jnp.float32),
          pltpu.VMEM((CH, C), jnp.int32),
          pltpu.VMEM((C, 16), jnp.float32),
          pltpu.VMEM((ZROWS, 16), jnp.float32),
          pltpu.SemaphoreType.DMA,
      ],
      compiler_params=pltpu.CompilerParams(use_tc_tiling_on_sc=False),
  )


def _pad128(x):
  return jnp.pad(x, ((0, 0), (0, 128 - x.shape[1])))


def _prep_body(degp_ref, h0_ref, dinv_ref, s0_ref):
  deg = degp_ref[0, :N, 0:1] + degp_ref[1, :N, 0:1] + 1.0
  dinv = lax.rsqrt(deg)
  dinv_ref[...] = dinv
  s0_ref[...] = jnp.pad(h0_ref[...], ((0, 0), (0, DP - D))) * dinv


def _layer_body(p_ref, s_ref, dinv_ref, w_ref, b_ref, g_ref, bt_ref, o_ref):
  dinv = dinv_ref[...]
  z = (p_ref[0, :N, :DP] + p_ref[1, :N, :DP] + s_ref[...]) * dinv
  h = jnp.dot(z, w_ref[...], preferred_element_type=jnp.float32) + b_ref[...]
  mu = jnp.mean(h, axis=0, keepdims=True)
  var = jnp.mean((h - mu) * (h - mu), axis=0, keepdims=True)
  hn = (h - mu) * lax.rsqrt(var + EPS) * g_ref[...] + bt_ref[...]
  o_ref[...] = jnp.maximum(hn, 0.0) * dinv


def _final_body(p_ref, s_ref, dinv_ref, wmu_ref, bmu_ref, wls_ref, bls_ref,
                mu_ref, ls_ref):
  z = (p_ref[0, :N, :DP] + p_ref[1, :N, :DP] + s_ref[...]) * dinv_ref[...]
  mu_ref[...] = (
      jnp.dot(z, wmu_ref[...], preferred_element_type=jnp.float32)
      + bmu_ref[...])
  ls_ref[...] = (
      jnp.dot(z, wls_ref[...], preferred_element_type=jnp.float32)
      + bls_ref[...])


_f32 = jnp.float32
_prep = pl.pallas_call(
    _prep_body,
    out_shape=(jax.ShapeDtypeStruct((N, 1), _f32),
               jax.ShapeDtypeStruct((N, DP), _f32)),
)
_layer = pl.pallas_call(
    _layer_body,
    out_shape=jax.ShapeDtypeStruct((N, DP), _f32),
)
_final = pl.pallas_call(
    _final_body,
    out_shape=(jax.ShapeDtypeStruct((N, OUT), _f32),
               jax.ShapeDtypeStruct((N, OUT), _f32)),
)


def _pad_w(w):
  return jnp.pad(w, ((0, DP - D), (0, max(0, DP - w.shape[1]))))


def _pad_row(v, width):
  return jnp.pad(v, (0, width - v.shape[0]))[None, :]


@jax.jit
def kernel(x, edge_index, emb, W0, b0, g0, bt0, W1, b1, g1, bt1,
           W2, b2, g2, bt2, W3, b3, g3, bt3, Wmu, bmu, Wls, bls):
  h0 = jnp.take(emb, x, axis=0, mode="clip")

  srcp = edge_index[0].reshape(NCH, C)
  dstp = edge_index[1].reshape(NCH, C)

  zrows = jnp.zeros((C, DP), _f32)
  ones16 = jnp.ones((C, 16), _f32)
  z16 = jnp.zeros((ZROWS, 16), _f32)

  degp = _get_deg()(dstp, ones16, z16)
  dinv, s = _prep(degp, h0)

  ws = [(_pad_w(W0), _pad_row(b0, DP), _pad_row(g0, DP), _pad_row(bt0, DP)),
        (_pad_w(W1), _pad_row(b1, DP), _pad_row(g1, DP), _pad_row(bt1, DP)),
        (_pad_w(W2), _pad_row(b2, DP), _pad_row(g2, DP), _pad_row(bt2, DP)),
        (_pad_w(W3), _pad_row(b3, DP), _pad_row(g3, DP), _pad_row(bt3, DP))]
  spmm = _get_spmm()
  for w, b, g, bt in ws:
    part = spmm(s, srcp, dstp, zrows)
    s = _layer(part, s, dinv, w, b, g, bt)

  part = spmm(s, srcp, dstp, zrows)
  mu, ls = _final(part, s, dinv, _pad_w(Wmu), _pad_row(bmu, OUT),
                  _pad_w(Wls), _pad_row(bls, OUT))
  return mu, ls
